# baseline (speedup 1.0000x reference)
import functools

import jax
import jax.numpy as jnp
from jax import lax
from jax.experimental import pallas as pl
from jax.experimental.pallas import tpu as pltpu
from jax.experimental.pallas import tpu_sc as plsc

S = 16
B = 32
H = 32
D = 256
G = 4 * H

VB = 8192


def _sc_gather(emb, idx_flat):
    info = plsc.get_sparse_core_info()
    nw = info.num_cores * info.num_subcores
    n = idx_flat.shape[0]
    b_per_w = n // nw
    mesh = plsc.VectorSubcoreMesh(core_axis_name="c", subcore_axis_name="s")

    @functools.partial(
        pl.kernel,
        mesh=mesh,
        out_type=jax.ShapeDtypeStruct((n, D), jnp.float32),
        scratch_types=[
            pltpu.VMEM((b_per_w,), jnp.int32),
            pltpu.VMEM((b_per_w, D), jnp.float32),
            pltpu.SemaphoreType.DMA,
        ],
    )
    def gather_kernel(table_hbm, idx_hbm, out_hbm, idx_v, rows_v, sem):
        wid = lax.axis_index("s") * info.num_cores + lax.axis_index("c")
        base = wid * b_per_w
        pltpu.sync_copy(idx_hbm.at[pl.ds(base, b_per_w)], idx_v)
        pltpu.async_copy(table_hbm.at[idx_v], rows_v, sem).wait()
        pltpu.sync_copy(rows_v, out_hbm.at[pl.ds(base, b_per_w)])

    return gather_kernel(emb, idx_flat)


def _lstm_fc_kernel(we_ref, wih_ref, whh_ref, b_ref, h0_ref, c0_ref,
                    wfc_ref, bfc_ref, logits_ref, hf_ref, cf_ref,
                    xw_scr, outs_scr):
    step_i = pl.program_id(0)

    @pl.when(step_i == 0)
    def _lstm():
        xw_scr[...] = lax.dot_general(
            we_ref[...], wih_ref[...], (((1,), (1,)), ((), ())),
            preferred_element_type=jnp.float32) + b_ref[...]

        def body(t, carry):
            h, c = carry
            gates = xw_scr[pl.ds(t * B, B), :] + lax.dot_general(
                h, whh_ref[...], (((1,), (1,)), ((), ())),
                preferred_element_type=jnp.float32)
            ig = jax.nn.sigmoid(gates[:, 0:H])
            fg = jax.nn.sigmoid(gates[:, H:2 * H])
            gg = jnp.tanh(gates[:, 2 * H:3 * H])
            og = jax.nn.sigmoid(gates[:, 3 * H:4 * H])
            c2 = fg * c + ig * gg
            h2 = og * jnp.tanh(c2)
            outs_scr[pl.ds(t * B, B), :] = h2
            return (h2, c2)

        hf, cf = lax.fori_loop(0, S, body, (h0_ref[...], c0_ref[...]))
        hf_ref[0, :, :] = hf
        cf_ref[0, :, :] = cf

    logits_ref[...] = lax.dot_general(
        outs_scr[...], wfc_ref[...], (((1,), (0,)), ((), ())),
        preferred_element_type=jnp.float32) + bfc_ref[...]


def kernel(x, h0, c0, emb, W_ih, W_hh, b_ih, b_hh, W_fc, b_fc):
    V = W_fc.shape[0]
    idx_flat = x.reshape(S * B).astype(jnp.int32)
    we = _sc_gather(emb, idx_flat)

    bias = (b_ih + b_hh).reshape(1, G)
    bfc2 = b_fc.reshape(1, V)
    wfcT = W_fc.T
    nv = pl.cdiv(V, VB)

    logits, hf, cf = pl.pallas_call(
        _lstm_fc_kernel,
        grid=(nv,),
        in_specs=[
            pl.BlockSpec((S * B, D), lambda i: (0, 0)),
            pl.BlockSpec((G, D), lambda i: (0, 0)),
            pl.BlockSpec((G, H), lambda i: (0, 0)),
            pl.BlockSpec((1, G), lambda i: (0, 0)),
            pl.BlockSpec((B, H), lambda i: (0, 0)),
            pl.BlockSpec((B, H), lambda i: (0, 0)),
            pl.BlockSpec((H, VB), lambda i: (0, i)),
            pl.BlockSpec((1, VB), lambda i: (0, i)),
        ],
        out_specs=[
            pl.BlockSpec((S * B, VB), lambda i: (0, i)),
            pl.BlockSpec((1, B, H), lambda i: (0, 0, 0)),
            pl.BlockSpec((1, B, H), lambda i: (0, 0, 0)),
        ],
        out_shape=[
            jax.ShapeDtypeStruct((S * B, V), jnp.float32),
            jax.ShapeDtypeStruct((1, B, H), jnp.float32),
            jax.ShapeDtypeStruct((1, B, H), jnp.float32),
        ],
        scratch_shapes=[
            pltpu.VMEM((S * B, G), jnp.float32),
            pltpu.VMEM((S * B, H), jnp.float32),
        ],
        compiler_params=pltpu.CompilerParams(
            dimension_semantics=("arbitrary",),
        ),
    )(we, W_ih, W_hh, bias, h0[0], c0[0], wfcT, bfc2)

    return logits.reshape(S, B, V), hf, cf


# speedup vs baseline: 1.0106x; 1.0106x over previous
import functools

import jax
import jax.numpy as jnp
from jax import lax
from jax.experimental import pallas as pl
from jax.experimental.pallas import tpu as pltpu
from jax.experimental.pallas import tpu_sc as plsc

S = 16
B = 32
H = 32
D = 256
G = 4 * H

VB = 4096


def _sc_gather(emb, idx_flat):
    info = plsc.get_sparse_core_info()
    nw = info.num_cores * info.num_subcores
    n = idx_flat.shape[0]
    b_per_w = n // nw
    mesh = plsc.VectorSubcoreMesh(core_axis_name="c", subcore_axis_name="s")

    @functools.partial(
        pl.kernel,
        mesh=mesh,
        out_type=jax.ShapeDtypeStruct((n, D), jnp.float32),
        scratch_types=[
            pltpu.VMEM((b_per_w,), jnp.int32),
            pltpu.VMEM((b_per_w, D), jnp.float32),
            pltpu.SemaphoreType.DMA,
        ],
    )
    def gather_kernel(table_hbm, idx_hbm, out_hbm, idx_v, rows_v, sem):
        wid = lax.axis_index("s") * info.num_cores + lax.axis_index("c")
        base = wid * b_per_w
        pltpu.sync_copy(idx_hbm.at[pl.ds(base, b_per_w)], idx_v)
        pltpu.async_copy(table_hbm.at[idx_v], rows_v, sem).wait()
        pltpu.sync_copy(rows_v, out_hbm.at[pl.ds(base, b_per_w)])

    return gather_kernel(emb, idx_flat)


def _lstm_fc_kernel(we_ref, wih_ref, whh_ref, b_ref, h0_ref, c0_ref,
                    wfc_ref, bfc_ref, logits_ref, hf_ref, cf_ref,
                    xw_scr, outs_scr):
    step_i = pl.program_id(0)

    @pl.when(step_i == 0)
    def _lstm():
        xw_scr[...] = lax.dot_general(
            we_ref[...], wih_ref[...], (((1,), (1,)), ((), ())),
            preferred_element_type=jnp.float32) + b_ref[...]

        def body(t, carry):
            h, c = carry
            gates = xw_scr[pl.ds(t * B, B), :] + lax.dot_general(
                h, whh_ref[...], (((1,), (1,)), ((), ())),
                preferred_element_type=jnp.float32)
            ig = jax.nn.sigmoid(gates[:, 0:H])
            fg = jax.nn.sigmoid(gates[:, H:2 * H])
            gg = jnp.tanh(gates[:, 2 * H:3 * H])
            og = jax.nn.sigmoid(gates[:, 3 * H:4 * H])
            c2 = fg * c + ig * gg
            h2 = og * jnp.tanh(c2)
            outs_scr[pl.ds(t * B, B), :] = h2
            return (h2, c2)

        hf, cf = lax.fori_loop(0, S, body, (h0_ref[...], c0_ref[...]))
        hf_ref[0, :, :] = hf
        cf_ref[0, :, :] = cf

    logits_ref[...] = lax.dot_general(
        outs_scr[...], wfc_ref[...], (((1,), (0,)), ((), ())),
        preferred_element_type=jnp.float32) + bfc_ref[...]


def kernel(x, h0, c0, emb, W_ih, W_hh, b_ih, b_hh, W_fc, b_fc):
    V = W_fc.shape[0]
    idx_flat = x.reshape(S * B).astype(jnp.int32)
    we = _sc_gather(emb, idx_flat)

    bias = (b_ih + b_hh).reshape(1, G)
    bfc2 = b_fc.reshape(1, V)
    wfcT = W_fc.T
    nv = pl.cdiv(V, VB)

    logits, hf, cf = pl.pallas_call(
        _lstm_fc_kernel,
        grid=(nv,),
        in_specs=[
            pl.BlockSpec((S * B, D), lambda i: (0, 0)),
            pl.BlockSpec((G, D), lambda i: (0, 0)),
            pl.BlockSpec((G, H), lambda i: (0, 0)),
            pl.BlockSpec((1, G), lambda i: (0, 0)),
            pl.BlockSpec((B, H), lambda i: (0, 0)),
            pl.BlockSpec((B, H), lambda i: (0, 0)),
            pl.BlockSpec((H, VB), lambda i: (0, i)),
            pl.BlockSpec((1, VB), lambda i: (0, i)),
        ],
        out_specs=[
            pl.BlockSpec((S * B, VB), lambda i: (0, i)),
            pl.BlockSpec((1, B, H), lambda i: (0, 0, 0)),
            pl.BlockSpec((1, B, H), lambda i: (0, 0, 0)),
        ],
        out_shape=[
            jax.ShapeDtypeStruct((S * B, V), jnp.float32),
            jax.ShapeDtypeStruct((1, B, H), jnp.float32),
            jax.ShapeDtypeStruct((1, B, H), jnp.float32),
        ],
        scratch_shapes=[
            pltpu.VMEM((S * B, G), jnp.float32),
            pltpu.VMEM((S * B, H), jnp.float32),
        ],
        compiler_params=pltpu.CompilerParams(
            dimension_semantics=("arbitrary",),
        ),
    )(we, W_ih, W_hh, bias, h0[0], c0[0], wfcT, bfc2)

    return logits.reshape(S, B, V), hf, cf


# speedup vs baseline: 1.1153x; 1.1036x over previous
import functools

import jax
import jax.numpy as jnp
from jax import lax
from jax.experimental import pallas as pl
from jax.experimental.pallas import tpu as pltpu
from jax.experimental.pallas import tpu_sc as plsc

S = 16
B = 32
H = 32
D = 256
G = 4 * H

VB = 4096


def _sc_gather(emb, idx_flat):
    info = plsc.get_sparse_core_info()
    nw = info.num_cores * info.num_subcores
    n = idx_flat.shape[0]
    b_per_w = n // nw
    mesh = plsc.VectorSubcoreMesh(core_axis_name="c", subcore_axis_name="s")

    @functools.partial(
        pl.kernel,
        mesh=mesh,
        out_type=jax.ShapeDtypeStruct((n, D), jnp.float32),
        scratch_types=[
            pltpu.VMEM((b_per_w,), jnp.int32),
            pltpu.VMEM((b_per_w, D), jnp.float32),
            pltpu.SemaphoreType.DMA,
        ],
    )
    def gather_kernel(table_hbm, idx_hbm, out_hbm, idx_v, rows_v, sem):
        wid = lax.axis_index("s") * info.num_cores + lax.axis_index("c")
        base = wid * b_per_w
        pltpu.sync_copy(idx_hbm.at[pl.ds(base, b_per_w)], idx_v)
        pltpu.async_copy(table_hbm.at[idx_v], rows_v, sem).wait()
        pltpu.sync_copy(rows_v, out_hbm.at[pl.ds(base, b_per_w)])

    return gather_kernel(emb, idx_flat)


def _lstm_fc_kernel(we_ref, wih_ref, whh_ref, b_ref, h0_ref, c0_ref,
                    wfc_ref, bfc_ref, logits_ref, hf_ref, cf_ref,
                    xw_scr, outs_scr):
    step_i = pl.program_id(0)

    @pl.when(step_i == 0)
    def _lstm():
        xw_scr[...] = lax.dot_general(
            we_ref[...], wih_ref[...], (((1,), (1,)), ((), ())),
            preferred_element_type=jnp.float32) + b_ref[...]

        def body(t, carry):
            h, c = carry
            gates = xw_scr[pl.ds(t * B, B), :] + lax.dot_general(
                h, whh_ref[...], (((1,), (1,)), ((), ())),
                preferred_element_type=jnp.float32)
            ig = jax.nn.sigmoid(gates[:, 0:H])
            fg = jax.nn.sigmoid(gates[:, H:2 * H])
            gg = jnp.tanh(gates[:, 2 * H:3 * H])
            og = jax.nn.sigmoid(gates[:, 3 * H:4 * H])
            c2 = fg * c + ig * gg
            h2 = og * jnp.tanh(c2)
            outs_scr[pl.ds(t * B, B), :] = h2
            return (h2, c2)

        hf, cf = lax.fori_loop(0, S, body, (h0_ref[...], c0_ref[...]))
        hf_ref[0, :, :] = hf
        cf_ref[0, :, :] = cf

    logits_ref[...] = lax.dot_general(
        outs_scr[...], wfc_ref[...], (((1,), (0,)), ((), ())),
        preferred_element_type=jnp.float32) + bfc_ref[...]


def kernel(x, h0, c0, emb, W_ih, W_hh, b_ih, b_hh, W_fc, b_fc):
    V = W_fc.shape[0]
    idx_flat = x.reshape(S * B).astype(jnp.int32)
    we = jnp.take(emb, idx_flat, axis=0)

    bias = (b_ih + b_hh).reshape(1, G)
    bfc2 = b_fc.reshape(1, V)
    wfcT = W_fc.T
    nv = pl.cdiv(V, VB)

    logits, hf, cf = pl.pallas_call(
        _lstm_fc_kernel,
        grid=(nv,),
        in_specs=[
            pl.BlockSpec((S * B, D), lambda i: (0, 0)),
            pl.BlockSpec((G, D), lambda i: (0, 0)),
            pl.BlockSpec((G, H), lambda i: (0, 0)),
            pl.BlockSpec((1, G), lambda i: (0, 0)),
            pl.BlockSpec((B, H), lambda i: (0, 0)),
            pl.BlockSpec((B, H), lambda i: (0, 0)),
            pl.BlockSpec((H, VB), lambda i: (0, i)),
            pl.BlockSpec((1, VB), lambda i: (0, i)),
        ],
        out_specs=[
            pl.BlockSpec((S * B, VB), lambda i: (0, i)),
            pl.BlockSpec((1, B, H), lambda i: (0, 0, 0)),
            pl.BlockSpec((1, B, H), lambda i: (0, 0, 0)),
        ],
        out_shape=[
            jax.ShapeDtypeStruct((S * B, V), jnp.float32),
            jax.ShapeDtypeStruct((1, B, H), jnp.float32),
            jax.ShapeDtypeStruct((1, B, H), jnp.float32),
        ],
        scratch_shapes=[
            pltpu.VMEM((S * B, G), jnp.float32),
            pltpu.VMEM((S * B, H), jnp.float32),
        ],
        compiler_params=pltpu.CompilerParams(
            dimension_semantics=("arbitrary",),
        ),
    )(we, W_ih, W_hh, bias, h0[0], c0[0], wfcT, bfc2)

    return logits.reshape(S, B, V), hf, cf
